# baseline (speedup 1.0000x reference)
import functools

import jax
import jax.numpy as jnp
from jax import lax
from jax.experimental import pallas as pl
from jax.experimental.pallas import tpu as pltpu
from jax.experimental.pallas import tpu_sc as plsc

B_, C_, N_, K_ = 2, 1024, 2048, 9
OUT_ = 1024
LPAD = 128
BNK = B_ * N_ * K_

NB_A = 512


def _proj_body(x_ref, wk_ref, bk_ref, wq_ref, bq_ref, wv_ref, bv_ref,
               knt_ref, qn_ref, vt_ref):
    xb = x_ref[...]
    kt = lax.dot_general(xb, wk_ref[...], (((0,), (1,)), ((), ())),
                         preferred_element_type=jnp.float32) + bk_ref[...]
    nk = jnp.sqrt(jnp.sum(kt * kt, axis=1, keepdims=True))
    knt_ref[...] = kt / jnp.maximum(nk, 1e-12)
    q = lax.dot_general(wq_ref[...], xb, (((1,), (0,)), ((), ())),
                        preferred_element_type=jnp.float32) + bq_ref[...]
    nq = jnp.sqrt(jnp.sum(q * q, axis=0, keepdims=True))
    qn_ref[...] = q / jnp.maximum(nq, 1e-12)
    vt = lax.dot_general(xb, wv_ref[...], (((0,), (1,)), ((), ())),
                         preferred_element_type=jnp.float32) + bv_ref[...]
    vt_ref[...] = vt.astype(jnp.bfloat16)


def _run_proj(x, wk, bk2, wq, bq2, wv, bv2):
    return pl.pallas_call(
        _proj_body,
        grid=(B_, N_ // NB_A),
        in_specs=[
            pl.BlockSpec((None, C_, NB_A), lambda b, i: (b, 0, i)),
            pl.BlockSpec((C_, C_), lambda b, i: (0, 0)),
            pl.BlockSpec((1, C_), lambda b, i: (0, 0)),
            pl.BlockSpec((C_, C_), lambda b, i: (0, 0)),
            pl.BlockSpec((C_, 1), lambda b, i: (0, 0)),
            pl.BlockSpec((C_, C_), lambda b, i: (0, 0)),
            pl.BlockSpec((1, C_), lambda b, i: (0, 0)),
        ],
        out_specs=[
            pl.BlockSpec((None, NB_A, C_), lambda b, i: (b, i, 0)),
            pl.BlockSpec((None, C_, NB_A), lambda b, i: (b, 0, i)),
            pl.BlockSpec((None, NB_A, C_), lambda b, i: (b, i, 0)),
        ],
        out_shape=[
            jax.ShapeDtypeStruct((B_, N_, C_), jnp.float32),
            jax.ShapeDtypeStruct((B_, C_, N_), jnp.float32),
            jax.ShapeDtypeStruct((B_, N_, C_), jnp.bfloat16),
        ],
    )(x, wk, bk2, wq, bq2, wv, bv2)


MB_B = 256


def _topk_body(knt_ref, qn_ref, w_ref, idx_ref):
    s = lax.dot_general(knt_ref[...], qn_ref[...], (((1,), (0,)), ((), ())),
                        preferred_element_type=jnp.float32)
    b = pl.program_id(0)
    jj = lax.broadcasted_iota(jnp.int32, (MB_B, N_), 1)
    col = lax.broadcasted_iota(jnp.int32, (MB_B, LPAD), 1)
    vals = jnp.zeros((MB_B, LPAD), jnp.float32)
    idxs = jnp.zeros((MB_B, LPAD), jnp.int32)
    neg = jnp.float32(-jnp.inf)
    for t in range(K_):
        mx = jnp.max(s, axis=1, keepdims=True)
        am = jnp.min(jnp.where(s == mx, jj, N_), axis=1, keepdims=True)
        vals = jnp.where(col == t, mx, vals)
        idxs = jnp.where(col == t, am, idxs)
        s = jnp.where(jj == am, neg, s)
    e = jnp.where(col < K_, jnp.exp(vals - vals[:, 0:1]), 0.0)
    w_ref[...] = e / jnp.sum(e, axis=1, keepdims=True)
    idx_ref[...] = jnp.where(col < K_, idxs + b * N_, 0)


def _run_topk(knt, qn):
    return pl.pallas_call(
        _topk_body,
        grid=(B_, N_ // MB_B),
        in_specs=[
            pl.BlockSpec((None, MB_B, C_), lambda b, i: (b, i, 0)),
            pl.BlockSpec((None, C_, N_), lambda b, i: (b, 0, 0)),
        ],
        out_specs=[
            pl.BlockSpec((None, MB_B, LPAD), lambda b, i: (b, i, 0)),
            pl.BlockSpec((None, MB_B, LPAD), lambda b, i: (b, i, 0)),
        ],
        out_shape=[
            jax.ShapeDtypeStruct((B_, N_, LPAD), jnp.float32),
            jax.ShapeDtypeStruct((B_, N_, LPAD), jnp.int32),
        ],
    )(knt, qn)


_NC, _NS = 2, 16
_NW = _NC * _NS
_RPW = BNK // _NW
_CH = 128
_NCHUNK = _RPW // _CH


def _gather_body(table_hbm, idx_hbm, out_hbm, idx_v, rows_v, sem):
    wid = lax.axis_index("s") * _NC + lax.axis_index("c")

    def chunk(i, carry):
        base = wid * _RPW + i * _CH
        pltpu.sync_copy(idx_hbm.at[pl.ds(base, _CH)], idx_v)
        pltpu.async_copy(table_hbm.at[idx_v], rows_v, sem).wait()
        pltpu.sync_copy(rows_v, out_hbm.at[pl.ds(base, _CH)])
        return carry

    lax.fori_loop(0, _NCHUNK, chunk, 0)


def _run_gather(table, gidx):
    fn = pl.kernel(
        _gather_body,
        mesh=plsc.VectorSubcoreMesh(core_axis_name="c", subcore_axis_name="s"),
        out_type=jax.ShapeDtypeStruct((BNK, C_ // 2), jnp.int32),
        scratch_types=[
            pltpu.VMEM((_CH,), jnp.int32),
            pltpu.VMEM((_CH, C_ // 2), jnp.int32),
            pltpu.SemaphoreType.DMA,
        ],
    )
    return fn(table, gidx)


MB_D = 128


def _agg_body(xg_ref, w_ref, wkt_ref, bc_ref, out_ref):
    wall = w_ref[...]
    col = lax.broadcasted_iota(jnp.int32, (MB_D, LPAD), 1)
    acc = bc_ref[...]
    for m in range(K_):
        scale = jnp.sum(jnp.where(col == m, wall, 0.0), axis=1, keepdims=True)
        xs = (xg_ref[:, m, :].astype(jnp.float32) * scale).astype(jnp.bfloat16)
        acc = acc + lax.dot_general(wkt_ref[m], xs, (((1,), (1,)), ((), ())),
                                    preferred_element_type=jnp.float32)
    out_ref[...] = acc


def _run_agg(xg4, w, wkt, bc2):
    return pl.pallas_call(
        _agg_body,
        grid=(B_, N_ // MB_D),
        in_specs=[
            pl.BlockSpec((None, MB_D, K_, C_), lambda b, i: (b, i, 0, 0)),
            pl.BlockSpec((None, MB_D, LPAD), lambda b, i: (b, i, 0)),
            pl.BlockSpec((K_, OUT_, C_), lambda b, i: (0, 0, 0)),
            pl.BlockSpec((OUT_, 1), lambda b, i: (0, 0)),
        ],
        out_specs=pl.BlockSpec((None, OUT_, MB_D), lambda b, i: (b, 0, i)),
        out_shape=jax.ShapeDtypeStruct((B_, OUT_, N_), jnp.float32),
    )(xg4, w, wkt, bc2)


def kernel(x, Wk, bk, Wq, bq, Wv, bv, Wconv, bconv):
    knt, qn, vt = _run_proj(x, Wk, bk.reshape(1, C_), Wq, bq.reshape(C_, 1),
                            Wv, bv.reshape(1, C_))
    w, idx = _run_topk(knt, qn)
    gidx = idx[:, :, :K_].reshape(BNK)
    table = lax.bitcast_convert_type(
        vt.reshape(B_ * N_, C_ // 2, 2), jnp.int32)
    xg_i32 = _run_gather(table, gidx)
    xg = lax.bitcast_convert_type(xg_i32, jnp.bfloat16).reshape(BNK, C_)
    wkt = jnp.transpose(Wconv, (2, 0, 1)).astype(jnp.bfloat16)
    return _run_agg(xg.reshape(B_, N_, K_, C_), w, wkt, bconv.reshape(OUT_, 1))


# speedup vs baseline: 4.4577x; 4.4577x over previous
import functools

import jax
import jax.numpy as jnp
from jax import lax
from jax.experimental import pallas as pl
from jax.experimental.pallas import tpu as pltpu
from jax.experimental.pallas import tpu_sc as plsc

B_, C_, N_, K_ = 2, 1024, 2048, 9
OUT_ = 1024
LPAD = 128
BNK = B_ * N_ * K_

NB_A = 512


def _proj_body(x_ref, wk_ref, bk_ref, wq_ref, bq_ref, wv_ref, bv_ref,
               knt_ref, qn_ref, vt_ref):
    xb = x_ref[...]
    kt = lax.dot_general(xb, wk_ref[...], (((0,), (1,)), ((), ())),
                         preferred_element_type=jnp.float32) + bk_ref[...]
    nk = jnp.sqrt(jnp.sum(kt * kt, axis=1, keepdims=True))
    knt_ref[...] = kt / jnp.maximum(nk, 1e-12)
    q = lax.dot_general(wq_ref[...], xb, (((1,), (0,)), ((), ())),
                        preferred_element_type=jnp.float32) + bq_ref[...]
    nq = jnp.sqrt(jnp.sum(q * q, axis=0, keepdims=True))
    qn_ref[...] = q / jnp.maximum(nq, 1e-12)
    vt_ref[...] = lax.dot_general(xb, wv_ref[...], (((0,), (1,)), ((), ())),
                                  preferred_element_type=jnp.float32) + bv_ref[...]


def _run_proj(x, wk, bk2, wq, bq2, wv, bv2):
    return pl.pallas_call(
        _proj_body,
        grid=(B_, N_ // NB_A),
        in_specs=[
            pl.BlockSpec((None, C_, NB_A), lambda b, i: (b, 0, i)),
            pl.BlockSpec((C_, C_), lambda b, i: (0, 0)),
            pl.BlockSpec((1, C_), lambda b, i: (0, 0)),
            pl.BlockSpec((C_, C_), lambda b, i: (0, 0)),
            pl.BlockSpec((C_, 1), lambda b, i: (0, 0)),
            pl.BlockSpec((C_, C_), lambda b, i: (0, 0)),
            pl.BlockSpec((1, C_), lambda b, i: (0, 0)),
        ],
        out_specs=[
            pl.BlockSpec((None, NB_A, C_), lambda b, i: (b, i, 0)),
            pl.BlockSpec((None, C_, NB_A), lambda b, i: (b, 0, i)),
            pl.BlockSpec((None, NB_A, C_), lambda b, i: (b, i, 0)),
        ],
        out_shape=[
            jax.ShapeDtypeStruct((B_, N_, C_), jnp.float32),
            jax.ShapeDtypeStruct((B_, C_, N_), jnp.float32),
            jax.ShapeDtypeStruct((B_, N_, C_), jnp.float32),
        ],
    )(x, wk, bk2, wq, bq2, wv, bv2)


MB_B = 256


def _topk_body(knt_ref, qn_ref, w_ref, idx_ref):
    s = lax.dot_general(knt_ref[...], qn_ref[...], (((1,), (0,)), ((), ())),
                        preferred_element_type=jnp.float32)
    b = pl.program_id(0)
    jj = lax.broadcasted_iota(jnp.int32, (MB_B, N_), 1)
    col = lax.broadcasted_iota(jnp.int32, (MB_B, LPAD), 1)
    vals = jnp.zeros((MB_B, LPAD), jnp.float32)
    idxs = jnp.zeros((MB_B, LPAD), jnp.int32)
    neg = jnp.float32(-jnp.inf)
    for t in range(K_):
        mx = jnp.max(s, axis=1, keepdims=True)
        am = jnp.min(jnp.where(s == mx, jj, N_), axis=1, keepdims=True)
        vals = jnp.where(col == t, mx, vals)
        idxs = jnp.where(col == t, am, idxs)
        s = jnp.where(jj == am, neg, s)
    e = jnp.where(col < K_, jnp.exp(vals - vals[:, 0:1]), 0.0)
    w_ref[...] = e / jnp.sum(e, axis=1, keepdims=True)
    idx_ref[...] = jnp.where(col < K_, idxs + b * N_, 0)


def _run_topk(knt, qn):
    return pl.pallas_call(
        _topk_body,
        grid=(B_, N_ // MB_B),
        in_specs=[
            pl.BlockSpec((None, MB_B, C_), lambda b, i: (b, i, 0)),
            pl.BlockSpec((None, C_, N_), lambda b, i: (b, 0, 0)),
        ],
        out_specs=[
            pl.BlockSpec((None, MB_B, LPAD), lambda b, i: (b, i, 0)),
            pl.BlockSpec((None, MB_B, LPAD), lambda b, i: (b, i, 0)),
        ],
        out_shape=[
            jax.ShapeDtypeStruct((B_, N_, LPAD), jnp.float32),
            jax.ShapeDtypeStruct((B_, N_, LPAD), jnp.int32),
        ],
    )(knt, qn)


_NC, _NS = 2, 16
_NW = _NC * _NS
_RPW = BNK // _NW
_CH = 64
_NCHUNK = _RPW // _CH


def _gather_body(table_hbm, idx_hbm, out_hbm, idx_v, rows_v, sem):
    wid = lax.axis_index("s") * _NC + lax.axis_index("c")

    def chunk(i, carry):
        base = wid * _RPW + i * _CH
        pltpu.sync_copy(idx_hbm.at[pl.ds(base, _CH)], idx_v)
        pltpu.async_copy(table_hbm.at[idx_v], rows_v, sem).wait()
        pltpu.sync_copy(rows_v, out_hbm.at[pl.ds(base, _CH)])
        return carry

    lax.fori_loop(0, _NCHUNK, chunk, 0)


def _run_gather(table, gidx):
    fn = pl.kernel(
        _gather_body,
        mesh=plsc.VectorSubcoreMesh(core_axis_name="c", subcore_axis_name="s"),
        out_type=jax.ShapeDtypeStruct((BNK, C_), jnp.float32),
        scratch_types=[
            pltpu.VMEM((_CH,), jnp.int32),
            pltpu.VMEM((_CH, C_), jnp.float32),
            pltpu.SemaphoreType.DMA,
        ],
    )
    return fn(table, gidx)


MB_D = 128


def _agg_body(xg_ref, w_ref, wkt_ref, bc_ref, out_ref):
    wall = w_ref[...]
    col = lax.broadcasted_iota(jnp.int32, (MB_D, LPAD), 1)
    acc = bc_ref[...]
    for m in range(K_):
        scale = jnp.sum(jnp.where(col == m, wall, 0.0), axis=1, keepdims=True)
        xs = (xg_ref[:, m, :].astype(jnp.float32) * scale).astype(jnp.bfloat16)
        acc = acc + lax.dot_general(wkt_ref[m], xs, (((1,), (1,)), ((), ())),
                                    preferred_element_type=jnp.float32)
    out_ref[...] = acc


def _run_agg(xg4, w, wkt, bc2):
    return pl.pallas_call(
        _agg_body,
        grid=(B_, N_ // MB_D),
        in_specs=[
            pl.BlockSpec((None, MB_D, K_, C_), lambda b, i: (b, i, 0, 0)),
            pl.BlockSpec((None, MB_D, LPAD), lambda b, i: (b, i, 0)),
            pl.BlockSpec((K_, OUT_, C_), lambda b, i: (0, 0, 0)),
            pl.BlockSpec((OUT_, 1), lambda b, i: (0, 0)),
        ],
        out_specs=pl.BlockSpec((None, OUT_, MB_D), lambda b, i: (b, 0, i)),
        out_shape=jax.ShapeDtypeStruct((B_, OUT_, N_), jnp.float32),
    )(xg4, w, wkt, bc2)


def kernel(x, Wk, bk, Wq, bq, Wv, bv, Wconv, bconv):
    knt, qn, vt = _run_proj(x, Wk, bk.reshape(1, C_), Wq, bq.reshape(C_, 1),
                            Wv, bv.reshape(1, C_))
    w, idx = _run_topk(knt, qn)
    gidx = idx[:, :, :K_].reshape(BNK)
    xg = _run_gather(vt.reshape(B_ * N_, C_), gidx)
    wkt = jnp.transpose(Wconv, (2, 0, 1)).astype(jnp.bfloat16)
    return _run_agg(xg.reshape(B_, N_, K_, C_), w, wkt, bconv.reshape(OUT_, 1))


# speedup vs baseline: 5.0744x; 1.1384x over previous
import functools

import jax
import jax.numpy as jnp
from jax import lax
from jax.experimental import pallas as pl
from jax.experimental.pallas import tpu as pltpu
from jax.experimental.pallas import tpu_sc as plsc

B_, C_, N_, K_ = 2, 1024, 2048, 9
OUT_ = 1024
LPAD = 128
BNK = B_ * N_ * K_

NB_A = 512


def _proj_body(x_ref, wk_ref, bk_ref, wq_ref, bq_ref, wv_ref, bv_ref,
               knt_ref, qn_ref, vt_ref):
    xb = x_ref[...]
    kt = lax.dot_general(xb, wk_ref[...], (((0,), (1,)), ((), ())),
                         preferred_element_type=jnp.float32) + bk_ref[...]
    nk = jnp.sqrt(jnp.sum(kt * kt, axis=1, keepdims=True))
    knt_ref[...] = kt / jnp.maximum(nk, 1e-12)
    q = lax.dot_general(wq_ref[...], xb, (((1,), (0,)), ((), ())),
                        preferred_element_type=jnp.float32) + bq_ref[...]
    nq = jnp.sqrt(jnp.sum(q * q, axis=0, keepdims=True))
    qn_ref[...] = q / jnp.maximum(nq, 1e-12)
    vt_ref[...] = lax.dot_general(xb, wv_ref[...], (((0,), (1,)), ((), ())),
                                  preferred_element_type=jnp.float32) + bv_ref[...]


def _run_proj(x, wk, bk2, wq, bq2, wv, bv2):
    return pl.pallas_call(
        _proj_body,
        grid=(B_, N_ // NB_A),
        in_specs=[
            pl.BlockSpec((None, C_, NB_A), lambda b, i: (b, 0, i)),
            pl.BlockSpec((C_, C_), lambda b, i: (0, 0)),
            pl.BlockSpec((1, C_), lambda b, i: (0, 0)),
            pl.BlockSpec((C_, C_), lambda b, i: (0, 0)),
            pl.BlockSpec((C_, 1), lambda b, i: (0, 0)),
            pl.BlockSpec((C_, C_), lambda b, i: (0, 0)),
            pl.BlockSpec((1, C_), lambda b, i: (0, 0)),
        ],
        out_specs=[
            pl.BlockSpec((None, NB_A, C_), lambda b, i: (b, i, 0)),
            pl.BlockSpec((None, C_, NB_A), lambda b, i: (b, 0, i)),
            pl.BlockSpec((None, NB_A, C_), lambda b, i: (b, i, 0)),
        ],
        out_shape=[
            jax.ShapeDtypeStruct((B_, N_, C_), jnp.float32),
            jax.ShapeDtypeStruct((B_, C_, N_), jnp.float32),
            jax.ShapeDtypeStruct((B_, N_, C_), jnp.float32),
        ],
    )(x, wk, bk2, wq, bq2, wv, bv2)


MB_B = 256


def _topk_body(knt_ref, qn_ref, w_ref, idx_ref):
    s = lax.dot_general(knt_ref[...], qn_ref[...], (((1,), (0,)), ((), ())),
                        preferred_element_type=jnp.float32)
    b = pl.program_id(0)
    jj = lax.broadcasted_iota(jnp.int32, (MB_B, N_), 1)
    col = lax.broadcasted_iota(jnp.int32, (MB_B, LPAD), 1)
    vals = jnp.zeros((MB_B, LPAD), jnp.float32)
    idxs = jnp.zeros((MB_B, LPAD), jnp.int32)
    neg = jnp.float32(-jnp.inf)
    for t in range(K_):
        mx = jnp.max(s, axis=1, keepdims=True)
        am = jnp.min(jnp.where(s == mx, jj, N_), axis=1, keepdims=True)
        vals = jnp.where(col == t, mx, vals)
        idxs = jnp.where(col == t, am, idxs)
        s = jnp.where(jj == am, neg, s)
    e = jnp.where(col < K_, jnp.exp(vals - vals[:, 0:1]), 0.0)
    w_ref[...] = e / jnp.sum(e, axis=1, keepdims=True)
    idx_ref[...] = jnp.where(col < K_, idxs + b * N_, 0)


def _run_topk(knt, qn):
    return pl.pallas_call(
        _topk_body,
        grid=(B_, N_ // MB_B),
        in_specs=[
            pl.BlockSpec((None, MB_B, C_), lambda b, i: (b, i, 0)),
            pl.BlockSpec((None, C_, N_), lambda b, i: (b, 0, 0)),
        ],
        out_specs=[
            pl.BlockSpec((None, MB_B, LPAD), lambda b, i: (b, i, 0)),
            pl.BlockSpec((None, MB_B, LPAD), lambda b, i: (b, i, 0)),
        ],
        out_shape=[
            jax.ShapeDtypeStruct((B_, N_, LPAD), jnp.float32),
            jax.ShapeDtypeStruct((B_, N_, LPAD), jnp.int32),
        ],
    )(knt, qn)


_NC, _NS = 2, 16
_NW = _NC * _NS
_RPW = BNK // _NW
_CH = 64
_NCHUNK = _RPW // _CH


def _gather_body(table_hbm, idx_hbm, out_hbm, idx_v, rows_v, sem):
    wid = lax.axis_index("s") * _NC + lax.axis_index("c")

    def chunk(i, carry):
        base = wid * _RPW + i * _CH
        pltpu.sync_copy(idx_hbm.at[pl.ds(base, _CH)], idx_v)
        pltpu.async_copy(table_hbm.at[idx_v], rows_v, sem).wait()
        pltpu.sync_copy(rows_v, out_hbm.at[pl.ds(base, _CH)])
        return carry

    lax.fori_loop(0, _NCHUNK, chunk, 0)


def _run_gather(table, gidx):
    fn = pl.kernel(
        _gather_body,
        mesh=plsc.VectorSubcoreMesh(core_axis_name="c", subcore_axis_name="s"),
        out_type=jax.ShapeDtypeStruct((BNK, C_), jnp.float32),
        scratch_types=[
            pltpu.VMEM((_CH,), jnp.int32),
            pltpu.VMEM((_CH, C_), jnp.float32),
            pltpu.SemaphoreType.DMA,
        ],
    )
    return fn(table, gidx)


MB_D = 128


def _agg_body(xg_ref, w_ref, wf_ref, bc_ref, out_ref):
    wall = w_ref[...]
    col = lax.broadcasted_iota(jnp.int32, (MB_D, LPAD), 1)
    parts = []
    for m in range(K_):
        scale = jnp.sum(jnp.where(col == m, wall, 0.0), axis=1, keepdims=True)
        parts.append(xg_ref[:, m, :] * scale)
    xs = jnp.concatenate(parts, axis=1)
    out_ref[...] = bc_ref[...] + lax.dot_general(
        wf_ref[...], xs, (((1,), (1,)), ((), ())),
        preferred_element_type=jnp.float32)


def _run_agg(xg4, w, wf, bc2):
    return pl.pallas_call(
        _agg_body,
        grid=(B_, N_ // MB_D),
        in_specs=[
            pl.BlockSpec((None, MB_D, K_, C_), lambda b, i: (b, i, 0, 0)),
            pl.BlockSpec((None, MB_D, LPAD), lambda b, i: (b, i, 0)),
            pl.BlockSpec((OUT_, K_ * C_), lambda b, i: (0, 0)),
            pl.BlockSpec((OUT_, 1), lambda b, i: (0, 0)),
        ],
        out_specs=pl.BlockSpec((None, OUT_, MB_D), lambda b, i: (b, 0, i)),
        out_shape=jax.ShapeDtypeStruct((B_, OUT_, N_), jnp.float32),
    )(xg4, w, wf, bc2)


def kernel(x, Wk, bk, Wq, bq, Wv, bv, Wconv, bconv):
    knt, qn, vt = _run_proj(x, Wk, bk.reshape(1, C_), Wq, bq.reshape(C_, 1),
                            Wv, bv.reshape(1, C_))
    w, idx = _run_topk(knt, qn)
    gidx = idx[:, :, :K_].reshape(BNK)
    xg = _run_gather(vt.reshape(B_ * N_, C_), gidx)
    wf = jnp.transpose(Wconv, (0, 2, 1)).reshape(OUT_, K_ * C_)
    return _run_agg(xg.reshape(B_, N_, K_, C_), w, wf, bconv.reshape(OUT_, 1))


# speedup vs baseline: 6.3571x; 1.2528x over previous
import functools

import jax
import jax.numpy as jnp
from jax import lax
from jax.experimental import pallas as pl
from jax.experimental.pallas import tpu as pltpu
from jax.experimental.pallas import tpu_sc as plsc

B_, C_, N_, K_ = 2, 1024, 2048, 9
OUT_ = 1024
LPAD = 128
BNK = B_ * N_ * K_

NB_A = 512


def _proj_body(x_ref, wk_ref, bk_ref, wq_ref, bq_ref, wv_ref, bv_ref,
               knt_ref, qn_ref, vt_ref):
    xb = x_ref[...]
    kt = lax.dot_general(xb, wk_ref[...], (((0,), (1,)), ((), ())),
                         preferred_element_type=jnp.float32) + bk_ref[...]
    nk = jnp.sqrt(jnp.sum(kt * kt, axis=1, keepdims=True))
    knt_ref[...] = kt / jnp.maximum(nk, 1e-12)
    q = lax.dot_general(wq_ref[...], xb, (((1,), (0,)), ((), ())),
                        preferred_element_type=jnp.float32) + bq_ref[...]
    nq = jnp.sqrt(jnp.sum(q * q, axis=0, keepdims=True))
    qn_ref[...] = q / jnp.maximum(nq, 1e-12)
    vt = lax.dot_general(xb, wv_ref[...], (((0,), (1,)), ((), ())),
                         preferred_element_type=jnp.float32) + bv_ref[...]
    ba = lax.bitcast_convert_type(vt[:, :C_ // 2], jnp.uint32)
    bb = lax.bitcast_convert_type(vt[:, C_ // 2:], jnp.uint32)
    ra = ba + jnp.uint32(0x7FFF) + ((ba >> jnp.uint32(16)) & jnp.uint32(1))
    rb = bb + jnp.uint32(0x7FFF) + ((bb >> jnp.uint32(16)) & jnp.uint32(1))
    packed = (ra >> jnp.uint32(16)) | (rb & jnp.uint32(0xFFFF0000))
    vt_ref[...] = lax.bitcast_convert_type(packed, jnp.int32)


def _run_proj(x, wk, bk2, wq, bq2, wv, bv2):
    return pl.pallas_call(
        _proj_body,
        grid=(B_, N_ // NB_A),
        in_specs=[
            pl.BlockSpec((None, C_, NB_A), lambda b, i: (b, 0, i)),
            pl.BlockSpec((C_, C_), lambda b, i: (0, 0)),
            pl.BlockSpec((1, C_), lambda b, i: (0, 0)),
            pl.BlockSpec((C_, C_), lambda b, i: (0, 0)),
            pl.BlockSpec((C_, 1), lambda b, i: (0, 0)),
            pl.BlockSpec((C_, C_), lambda b, i: (0, 0)),
            pl.BlockSpec((1, C_), lambda b, i: (0, 0)),
        ],
        out_specs=[
            pl.BlockSpec((None, NB_A, C_), lambda b, i: (b, i, 0)),
            pl.BlockSpec((None, C_, NB_A), lambda b, i: (b, 0, i)),
            pl.BlockSpec((None, NB_A, C_ // 2), lambda b, i: (b, i, 0)),
        ],
        out_shape=[
            jax.ShapeDtypeStruct((B_, N_, C_), jnp.float32),
            jax.ShapeDtypeStruct((B_, C_, N_), jnp.float32),
            jax.ShapeDtypeStruct((B_, N_, C_ // 2), jnp.int32),
        ],
    )(x, wk, bk2, wq, bq2, wv, bv2)


MB_B = 256


def _topk_body(knt_ref, qn_ref, w_ref, idx_ref):
    s = lax.dot_general(knt_ref[...], qn_ref[...], (((1,), (0,)), ((), ())),
                        preferred_element_type=jnp.float32)
    b = pl.program_id(0)
    jj = lax.broadcasted_iota(jnp.int32, (MB_B, N_), 1)
    col = lax.broadcasted_iota(jnp.int32, (MB_B, LPAD), 1)
    vals = jnp.zeros((MB_B, LPAD), jnp.float32)
    idxs = jnp.zeros((MB_B, LPAD), jnp.int32)
    neg = jnp.float32(-jnp.inf)
    for t in range(K_):
        mx = jnp.max(s, axis=1, keepdims=True)
        am = jnp.min(jnp.where(s == mx, jj, N_), axis=1, keepdims=True)
        vals = jnp.where(col == t, mx, vals)
        idxs = jnp.where(col == t, am, idxs)
        s = jnp.where(jj == am, neg, s)
    e = jnp.where(col < K_, jnp.exp(vals - vals[:, 0:1]), 0.0)
    w_ref[...] = e / jnp.sum(e, axis=1, keepdims=True)
    idx_ref[...] = jnp.where(col < K_, idxs + b * N_, 0)


def _run_topk(knt, qn):
    return pl.pallas_call(
        _topk_body,
        grid=(B_, N_ // MB_B),
        in_specs=[
            pl.BlockSpec((None, MB_B, C_), lambda b, i: (b, i, 0)),
            pl.BlockSpec((None, C_, N_), lambda b, i: (b, 0, 0)),
        ],
        out_specs=[
            pl.BlockSpec((None, MB_B, LPAD), lambda b, i: (b, i, 0)),
            pl.BlockSpec((None, MB_B, LPAD), lambda b, i: (b, i, 0)),
        ],
        out_shape=[
            jax.ShapeDtypeStruct((B_, N_, LPAD), jnp.float32),
            jax.ShapeDtypeStruct((B_, N_, LPAD), jnp.int32),
        ],
    )(knt, qn)


_NC, _NS = 2, 16
_NW = _NC * _NS
_RPW = BNK // _NW
_CH = 128
_NCHUNK = _RPW // _CH


def _gather_body(table_hbm, idx_hbm, out_hbm, idx_v, rows_v, sem):
    wid = lax.axis_index("s") * _NC + lax.axis_index("c")

    def chunk(i, carry):
        base = wid * _RPW + i * _CH
        pltpu.sync_copy(idx_hbm.at[pl.ds(base, _CH)], idx_v)
        pltpu.async_copy(table_hbm.at[idx_v], rows_v, sem).wait()
        pltpu.sync_copy(rows_v, out_hbm.at[pl.ds(base, _CH)])
        return carry

    lax.fori_loop(0, _NCHUNK, chunk, 0)


def _run_gather(table, gidx):
    fn = pl.kernel(
        _gather_body,
        mesh=plsc.VectorSubcoreMesh(core_axis_name="c", subcore_axis_name="s"),
        out_type=jax.ShapeDtypeStruct((BNK, C_ // 2), jnp.int32),
        scratch_types=[
            pltpu.VMEM((_CH,), jnp.int32),
            pltpu.VMEM((_CH, C_ // 2), jnp.int32),
            pltpu.SemaphoreType.DMA,
        ],
    )
    return fn(table, gidx)


MB_D = 256


def _agg_body(xg_ref, w_ref, wf_ref, bc_ref, out_ref):
    wall = w_ref[...]
    col = lax.broadcasted_iota(jnp.int32, (MB_D, LPAD), 1)
    parts = []
    for m in range(K_):
        scale = jnp.sum(jnp.where(col == m, wall, 0.0), axis=1, keepdims=True)
        xu = lax.bitcast_convert_type(xg_ref[:, m, :], jnp.uint32)
        fe = lax.bitcast_convert_type(xu << jnp.uint32(16), jnp.float32)
        fo = lax.bitcast_convert_type(xu & jnp.uint32(0xFFFF0000), jnp.float32)
        parts.append(fe * scale)
        parts.append(fo * scale)
    xs = jnp.concatenate(parts, axis=1)
    out_ref[...] = bc_ref[...] + lax.dot_general(
        wf_ref[...], xs, (((1,), (1,)), ((), ())),
        preferred_element_type=jnp.float32)


def _run_agg(xg4, w, wf, bc2):
    return pl.pallas_call(
        _agg_body,
        grid=(B_, N_ // MB_D),
        in_specs=[
            pl.BlockSpec((None, MB_D, K_, C_ // 2), lambda b, i: (b, i, 0, 0)),
            pl.BlockSpec((None, MB_D, LPAD), lambda b, i: (b, i, 0)),
            pl.BlockSpec((OUT_, K_ * C_), lambda b, i: (0, 0)),
            pl.BlockSpec((OUT_, 1), lambda b, i: (0, 0)),
        ],
        out_specs=pl.BlockSpec((None, OUT_, MB_D), lambda b, i: (b, 0, i)),
        out_shape=jax.ShapeDtypeStruct((B_, OUT_, N_), jnp.float32),
    )(xg4, w, wf, bc2)


def kernel(x, Wk, bk, Wq, bq, Wv, bv, Wconv, bconv):
    knt, qn, vt = _run_proj(x, Wk, bk.reshape(1, C_), Wq, bq.reshape(C_, 1),
                            Wv, bv.reshape(1, C_))
    w, idx = _run_topk(knt, qn)
    gidx = idx[:, :, :K_].reshape(BNK)
    xg = _run_gather(vt.reshape(B_ * N_, C_ // 2), gidx)
    wf = jnp.transpose(Wconv, (0, 2, 1)).reshape(OUT_, K_ * C_)
    return _run_agg(xg.reshape(B_, N_, K_, C_ // 2), w, wf,
                    bconv.reshape(OUT_, 1))
